# Initial kernel scaffold; baseline (speedup 1.0000x reference)
import jax
import jax.numpy as jnp
from jax.experimental import pallas as pl


def kernel(x, func_vars, cpd):
    raise NotImplementedError("write your pallas kernel here")


# speedup vs baseline: 11.5991x; 11.5991x over previous
import numpy as np
import jax
import jax.numpy as jnp
from jax import lax
from jax.experimental import pallas as pl

_K = 4
_NC = 1 << _K


def _moebius_t():
    ss = lax.broadcasted_iota(jnp.int32, (_NC, _NC), 0)
    jj = lax.broadcasted_iota(jnp.int32, (_NC, _NC), 1)
    subset = (jj & ss) == jj
    d = ss ^ jj
    pc = (d & 1) + ((d >> 1) & 1) + ((d >> 2) & 1) + ((d >> 3) & 1)
    sign = (1 - 2 * (pc & 1)).astype(jnp.float32)
    return jnp.where(subset, sign, 0.0)


def _body(x_ref, fvt_ref, cpdt_ref, out_ref):
    bb, v = x_ref.shape
    tk = fvt_ref.shape[1]
    t = tk // _K

    iota_v = lax.broadcasted_iota(jnp.int32, (v, tk), 0)
    sel = (iota_v == fvt_ref[...]).astype(jnp.float32)
    g = jnp.dot(x_ref[...], sel, preferred_element_type=jnp.float32)

    g0 = g[:, 0 * t:1 * t]
    g1 = g[:, 1 * t:2 * t]
    g2 = g[:, 2 * t:3 * t]
    g3 = g[:, 3 * t:4 * t]

    cp = jnp.dot(_moebius_t(), cpdt_ref[...],
                 preferred_element_type=jnp.float32)

    q3 = g2 * g3
    p3 = g0 * g1

    def r(i):
        return (cp[4 * i + 0][None, :]
                + cp[4 * i + 1][None, :] * g3
                + cp[4 * i + 2][None, :] * g2
                + cp[4 * i + 3][None, :] * q3)

    inner = r(0) + r(1) * g1 + r(2) * g0 + r(3) * p3
    out_ref[...] = jnp.sum(inner, axis=1, keepdims=True)


def kernel(x, func_vars, cpd):
    b, v = x.shape
    t, k = func_vars.shape
    assert k == _K
    bb = 1024
    fvt = func_vars.T.reshape(1, t * k).astype(jnp.int32)
    cpdt = cpd.T

    out = pl.pallas_call(
        _body,
        grid=(b // bb,),
        in_specs=[
            pl.BlockSpec((bb, v), lambda i: (i, 0)),
            pl.BlockSpec((1, t * k), lambda i: (0, 0)),
            pl.BlockSpec((_NC, t), lambda i: (0, 0)),
        ],
        out_specs=pl.BlockSpec((bb, 1), lambda i: (i, 0)),
        out_shape=jax.ShapeDtypeStruct((b, 1), jnp.float32),
    )(x, fvt, cpdt)
    return out.reshape(b)
